# baseline (speedup 1.0000x reference)
import functools

import jax
import jax.numpy as jnp
from jax import lax
from jax.experimental import pallas as pl
from jax.experimental.pallas import tpu as pltpu
from jax.experimental.pallas import tpu_sc as plsc

BATCH = 16384
LATENT = 32
NC = 2
NS = 16
L = 16
NW = NC * NS
BPW = BATCH // NW
CHUNK = 128
NCH = BPW // CHUNK


def _mf_body(u_idx, i_idx, u_emb, i_emb, out,
             uidx_v, iidx_v, urows_v, irows_v, out_v, sem):
    wid = lax.axis_index("s") * NC + lax.axis_index("c")
    base = wid * BPW

    for ch in range(NCH):
        pltpu.sync_copy(u_idx.at[pl.ds(base + ch * CHUNK, CHUNK)], uidx_v.at[ch])
        pltpu.sync_copy(i_idx.at[pl.ds(base + ch * CHUNK, CHUNK)], iidx_v.at[ch])

    cps = []
    for ch in range(NCH):
        cps.append(pltpu.async_copy(
            u_emb.at[uidx_v.at[ch]], urows_v.at[pl.ds(ch * CHUNK, CHUNK)], sem))
        cps.append(pltpu.async_copy(
            i_emb.at[iidx_v.at[ch]], irows_v.at[pl.ds(ch * CHUNK, CHUNK)], sem))
    for cp in cps:
        cp.wait()

    def group(g, carry):
        rows = lax.iota(jnp.int32, L) + g * L
        acc = jnp.zeros((L,), jnp.float32)
        for d in range(LATENT):
            cols = jnp.full((L,), d, jnp.int32)
            uu = plsc.load_gather(urows_v, [rows, cols])
            ii = plsc.load_gather(irows_v, [rows, cols])
            acc = acc + uu * ii
        out_v[pl.ds(g * L, L)] = acc
        return carry

    lax.fori_loop(0, BPW // L, group, 0)

    pltpu.sync_copy(out_v, out.at[pl.ds(base, BPW)])


@functools.partial(jax.jit)
def kernel(u_idx, i_idx, u_emb, i_emb):
    mesh = plsc.VectorSubcoreMesh(core_axis_name="c", subcore_axis_name="s")
    f = pl.kernel(
        _mf_body,
        out_type=jax.ShapeDtypeStruct((BATCH,), jnp.float32),
        mesh=mesh,
        scratch_types=[
            pltpu.VMEM((NCH, CHUNK), jnp.int32),
            pltpu.VMEM((NCH, CHUNK), jnp.int32),
            pltpu.VMEM((BPW, LATENT), jnp.float32),
            pltpu.VMEM((BPW, LATENT), jnp.float32),
            pltpu.VMEM((BPW,), jnp.float32),
            pltpu.SemaphoreType.DMA,
        ],
        compiler_params=pltpu.CompilerParams(
            needs_layout_passes=False, use_tc_tiling_on_sc=False),
    )
    return f(u_idx, i_idx, u_emb, i_emb)


# speedup vs baseline: 1.4976x; 1.4976x over previous
import functools

import jax
import jax.numpy as jnp
from jax import lax
from jax.experimental import pallas as pl
from jax.experimental.pallas import tpu as pltpu
from jax.experimental.pallas import tpu_sc as plsc

BATCH = 16384
LATENT = 32
NC = 2
NS = 16
L = 16
NW = NC * NS
BPW = BATCH // NW
CH = 128
NCHK = BPW // CH
GPC = CH // L


def _mf_body(u_idx, i_idx, u_emb, i_emb, out,
             uidx_v, iidx_v, ubuf, ibuf, out_v, sems):
    wid = lax.axis_index("s") * NC + lax.axis_index("c")
    base = wid * BPW

    pltpu.sync_copy(u_idx.at[pl.ds(base, BPW)], uidx_v)
    pltpu.sync_copy(i_idx.at[pl.ds(base, BPW)], iidx_v)

    def fire_chunk(k, p):
        def fire(g, c):
            uvec = uidx_v[pl.ds(k * CH + g * L, L)]
            ivec = iidx_v[pl.ds(k * CH + g * L, L)]
            for j in range(L):
                b = g * L + j
                pltpu.async_copy(
                    u_emb.at[pl.ds(uvec[j], 1)],
                    ubuf.at[p].at[pl.ds(b, 1)], sems.at[0, p])
                pltpu.async_copy(
                    i_emb.at[pl.ds(ivec[j], 1)],
                    ibuf.at[p].at[pl.ds(b, 1)], sems.at[1, p])
            return c
        lax.fori_loop(0, GPC, fire, 0)

    def drain_chunk(p):
        pltpu.make_async_copy(
            u_emb.at[pl.ds(0, CH)], ubuf.at[p], sems.at[0, p]).wait()
        pltpu.make_async_copy(
            i_emb.at[pl.ds(0, CH)], ibuf.at[p], sems.at[1, p]).wait()

    fire_chunk(0, 0)
    for k in range(NCHK):
        p = k % 2
        if k + 1 < NCHK:
            fire_chunk(k + 1, (k + 1) % 2)
        drain_chunk(p)

        def group(g, c):
            rows = lax.iota(jnp.int32, L) + g * L
            acc = jnp.zeros((L,), jnp.float32)
            for d in range(LATENT):
                cols = jnp.full((L,), d, jnp.int32)
                uu = plsc.load_gather(ubuf.at[p], [rows, cols])
                ii = plsc.load_gather(ibuf.at[p], [rows, cols])
                acc = acc + uu * ii
            out_v[pl.ds(k * CH + g * L, L)] = acc
            return c

        lax.fori_loop(0, GPC, group, 0)

    pltpu.sync_copy(out_v, out.at[pl.ds(base, BPW)])


@functools.partial(jax.jit)
def kernel(u_idx, i_idx, u_emb, i_emb):
    mesh = plsc.VectorSubcoreMesh(core_axis_name="c", subcore_axis_name="s")
    f = pl.kernel(
        _mf_body,
        out_type=jax.ShapeDtypeStruct((BATCH,), jnp.float32),
        mesh=mesh,
        scratch_types=[
            pltpu.VMEM((BPW,), jnp.int32),
            pltpu.VMEM((BPW,), jnp.int32),
            pltpu.VMEM((2, CH, LATENT), jnp.float32),
            pltpu.VMEM((2, CH, LATENT), jnp.float32),
            pltpu.VMEM((BPW,), jnp.float32),
            pltpu.SemaphoreType.DMA((2, 2)),
        ],
        compiler_params=pltpu.CompilerParams(needs_layout_passes=False),
    )
    return f(u_idx, i_idx, u_emb, i_emb)
